# baseline (speedup 1.0000x reference)
import functools

import jax
import jax.numpy as jnp
from jax import lax
from jax.experimental import pallas as pl
from jax.experimental.pallas import tpu as pltpu
from jax.experimental.pallas import tpu_sc as plsc

N_NODES = 10000
N_EDGES = 320000
D = 128

NC = 2
NS = 16
NPAD = 10240
RPT = NPAD // NS
NROW = N_NODES // NS
EPC = N_EDGES // NC
EPT = EPC // NS
C = 80
NCHUNK_D = EPT // C
EPT_P = EPT
NCHUNK = EPT_P // C
KB = 25
NB = NCHUNK // KB

def _zero_rows(buf, nrows):
    zv = jnp.zeros((16,), jnp.float32)

    def body(i, _):
        for j in range(D // 16):
            buf[i, pl.ds(j * 16, 16)] = zv
        return 0

    lax.fori_loop(0, nrows, body, 0)


def _sc_deg_body(dst4_hbm, ew4_hbm, out_hbm, di_v, w_v, zb_v, dsem, deg_sh):
    c = lax.axis_index("c")
    s = lax.axis_index("s")
    pltpu.sync_copy(dst4_hbm.at[c, s], di_v)
    pltpu.sync_copy(ew4_hbm.at[c, s], w_v)
    zv = jnp.zeros((16,), jnp.float32)

    def zb(i, _):
        zb_v[pl.ds(i * 16, 16)] = zv
        return 0

    lax.fori_loop(0, RPT // 16, zb, 0)
    pltpu.sync_copy(zb_v, deg_sh.at[pl.ds(s * RPT, RPT)])
    plsc.subcore_barrier()

    def fire(k, _):
        pltpu.async_copy(w_v.at[k], deg_sh.at[di_v.at[k]], dsem, add=True)
        return 0

    lax.fori_loop(0, NCHUNK_D, fire, 0)

    def drain(k, _):
        pltpu.make_async_copy(w_v.at[0], deg_sh.at[di_v.at[0]], dsem).wait()
        return 0

    lax.fori_loop(0, NCHUNK_D, drain, 0)
    plsc.subcore_barrier()
    pltpu.sync_copy(deg_sh.at[pl.ds(s * RPT, RPT)],
                    out_hbm.at[c, pl.ds(s * RPT, RPT)])


def _sc_edge_body(ep_hbm, ew5_hbm, hs_hbm, out_hbm, e_v, w_v, rows0, rows1,
                  rows2, gs0, gs1, gs2, ss0, ss1, ss2, acc_sh):
    c = lax.axis_index("c")
    s = lax.axis_index("s")
    rows = (rows0, rows1, rows2)
    gs = (gs0, gs1, gs2)
    ss = (ss0, ss1, ss2)
    _zero_rows(rows0, C)
    for i in range(RPT // C):
        pltpu.sync_copy(rows0, acc_sh.at[pl.ds(s * RPT + i * C, C)])
    plsc.subcore_barrier()

    def scale(buf, k):
        @plsc.parallel_loop(0, C // 16, step=1)
        def grp(g):
            wv = w_v[k, pl.ds(g * 16, 16)]
            for l in range(16):
                wl = wv[l]
                r = g * 16 + l
                for j in range(D // 16):
                    buf[r, pl.ds(j * 16, 16)] = buf[r, pl.ds(j * 16, 16)] * wl

    def gstart(k, j):
        pltpu.async_copy(hs_hbm.at[e_v.at[0, k]], rows[j], gs[j])

    def gwait(j):
        pltpu.make_async_copy(hs_hbm.at[e_v.at[0, 0]], rows[j], gs[j]).wait()

    def sstart(k, j):
        pltpu.async_copy(rows[j], acc_sh.at[e_v.at[1, k]], ss[j], add=True)

    def swait(j):
        pltpu.make_async_copy(rows[j], acc_sh.at[e_v.at[1, 0]], ss[j]).wait()

    def step(k, j):
        gwait(j)
        scale(rows[j], k)
        sstart(k, j)
        prv = (j - 1) % 3

        @pl.when(k > 0)
        def _():
            swait(prv)

        @pl.when(k + 2 < KB)
        def _():
            gstart(k + 2, prv)

    def block(b, _):
        pltpu.sync_copy(ep_hbm.at[c, s, b], e_v)
        pltpu.sync_copy(ew5_hbm.at[c, s, b], w_v)
        gstart(0, 0)
        gstart(1, 1)

        def tri(i, _):
            for j in range(3):
                step(3 * i + j, j)
            return 0

        lax.fori_loop(0, KB // 3, tri, 0)
        gwait(0)
        scale(rows0, KB - 1)
        sstart(KB - 1, 0)
        swait(2)
        swait(0)
        return 0

    lax.fori_loop(0, NB, block, 0)
    plsc.subcore_barrier()
    pltpu.sync_copy(acc_sh.at[pl.ds(s * RPT, RPT)],
                    out_hbm.at[c, pl.ds(s * RPT, RPT)])


@functools.cache
def _sc_kernels():
    mesh = plsc.VectorSubcoreMesh(core_axis_name="c", subcore_axis_name="s",
                                  num_cores=NC, num_subcores=NS)
    sc_deg = pl.kernel(
        _sc_deg_body,
        out_type=jax.ShapeDtypeStruct((NC, NPAD), jnp.float32),
        mesh=mesh,
        scratch_types=[
            pltpu.VMEM((NCHUNK_D, C), jnp.int32),
            pltpu.VMEM((NCHUNK_D, C), jnp.float32),
            pltpu.VMEM((RPT,), jnp.float32),
            pltpu.SemaphoreType.DMA,
            pltpu.VMEM_SHARED((NPAD,), jnp.float32),
        ],
    )
    sc_edge = pl.kernel(
        _sc_edge_body,
        out_type=jax.ShapeDtypeStruct((NC, NPAD, D), jnp.float32),
        mesh=mesh,
        scratch_types=[
            pltpu.VMEM((2, KB, C), jnp.int32),
            pltpu.VMEM((KB, C), jnp.float32),
            pltpu.VMEM((C, D), jnp.float32),
            pltpu.VMEM((C, D), jnp.float32),
            pltpu.VMEM((C, D), jnp.float32),
            pltpu.SemaphoreType.DMA,
            pltpu.SemaphoreType.DMA,
            pltpu.SemaphoreType.DMA,
            pltpu.SemaphoreType.DMA,
            pltpu.SemaphoreType.DMA,
            pltpu.SemaphoreType.DMA,
            pltpu.VMEM_SHARED((NPAD, D), jnp.float32),
        ],
    )
    return sc_deg, sc_edge


_RB = 1000
_GRID = N_NODES // _RB

_BLK = pl.BlockSpec((_RB, D), lambda i: (i, 0))
_BLK_ACC = pl.BlockSpec((NC, _RB, D), lambda i: (0, i, 0))
_BLK_DEG = pl.BlockSpec((NC, _RB, 1), lambda i: (0, i, 0))
_BLK_BIAS = pl.BlockSpec((1, D), lambda i: (0, 0))
_BLK_W = pl.BlockSpec((D, D), lambda i: (0, 0))
_OUT_SHAPE = jax.ShapeDtypeStruct((N_NODES, D), jnp.float32)


def _dinv(dp_ref):
    return lax.rsqrt(dp_ref[0] + dp_ref[1] + 1.0)


def _tc_mm_scale_body(x_ref, w_ref, dp_ref, o_ref):
    o_ref[...] = jnp.dot(x_ref[...], w_ref[...],
                         preferred_element_type=jnp.float32) * _dinv(dp_ref)


def _tc_mm_scale(x, W, degp):
    return pl.pallas_call(
        _tc_mm_scale_body,
        grid=(_GRID,),
        in_specs=[_BLK, _BLK_W, _BLK_DEG],
        out_specs=_BLK,
        out_shape=_OUT_SHAPE,
    )(x, W, degp)


def _tc_mid_body(acc_ref, hs_ref, dp_ref, bias_ref, w_ref, o_ref):
    dv = _dinv(dp_ref)
    t = (acc_ref[0] + acc_ref[1] + hs_ref[...]) * dv + bias_ref[...]
    h = jnp.maximum(t, 0.0)
    o_ref[...] = jnp.dot(h, w_ref[...],
                         preferred_element_type=jnp.float32) * dv


def _tc_mid(acc, hs, degp, bias, W):
    return pl.pallas_call(
        _tc_mid_body,
        grid=(_GRID,),
        in_specs=[_BLK_ACC, _BLK, _BLK_DEG, _BLK_BIAS, _BLK_W],
        out_specs=_BLK,
        out_shape=_OUT_SHAPE,
    )(acc, hs, degp, bias, W)


def _tc_out_body(acc_ref, hs_ref, dp_ref, bias_ref, o_ref):
    o_ref[...] = (acc_ref[0] + acc_ref[1] + hs_ref[...]) * _dinv(dp_ref) \
        + bias_ref[...]


def _tc_out(acc, hs, degp, bias):
    return pl.pallas_call(
        _tc_out_body,
        grid=(_GRID,),
        in_specs=[_BLK_ACC, _BLK, _BLK_DEG, _BLK_BIAS],
        out_specs=_BLK,
        out_shape=_OUT_SHAPE,
    )(acc, hs, degp, bias)


def kernel(x, edge_index, edge_weight, W1, b1, W2, b2):
    src = edge_index[0]
    dst = edge_index[1]

    ep = jnp.stack([src.reshape(NC, NS, NB, KB, C),
                    dst.reshape(NC, NS, NB, KB, C)], axis=3)
    ew5 = edge_weight.reshape(NC, NS, NB, KB, C)
    dst4 = dst.reshape(NC, NS, NCHUNK_D, C)
    ew4 = edge_weight.reshape(NC, NS, NCHUNK_D, C)
    _sc_deg, _sc_edge = _sc_kernels()
    degp = _sc_deg(dst4, ew4)
    dp = degp.reshape(NC, NPAD, 1)
    hs1 = _tc_mm_scale(x, W1, dp)
    acc1 = _sc_edge(ep, ew5, hs1)
    hs2 = _tc_mid(acc1, hs1, dp, b1.reshape(1, D), W2)
    acc2 = _sc_edge(ep, ew5, hs2)
    return _tc_out(acc2, hs2, dp, b2.reshape(1, D))


# speedup vs baseline: 1.1636x; 1.1636x over previous
import functools

import jax
import jax.numpy as jnp
from jax import lax
from jax.experimental import pallas as pl
from jax.experimental.pallas import tpu as pltpu
from jax.experimental.pallas import tpu_sc as plsc

N_NODES = 10000
N_EDGES = 320000
D = 128

NC = 2
NS = 16
NPAD = 10240
RPT = NPAD // NS
NROW = N_NODES // NS
EPC = N_EDGES // NC
EPT = EPC // NS
C = 80
NCHUNK_D = EPT // C
EPT_P = EPT
NCHUNK = EPT_P // C
KB = 25
NB = NCHUNK // KB

def _zero_rows(buf, nrows):
    zv = jnp.zeros((16,), jnp.float32)

    def body(i, _):
        for j in range(D // 16):
            buf[i, pl.ds(j * 16, 16)] = zv
        return 0

    lax.fori_loop(0, nrows, body, 0)


def _sc_deg_body(dst4_hbm, ew4_hbm, out_hbm, di_v, w_v, zb_v, dsem, deg_sh):
    c = lax.axis_index("c")
    s = lax.axis_index("s")
    pltpu.sync_copy(dst4_hbm.at[c, s], di_v)
    pltpu.sync_copy(ew4_hbm.at[c, s], w_v)
    zv = jnp.zeros((16,), jnp.float32)

    def zb(i, _):
        zb_v[pl.ds(i * 16, 16)] = zv
        return 0

    lax.fori_loop(0, RPT // 16, zb, 0)
    pltpu.sync_copy(zb_v, deg_sh.at[pl.ds(s * RPT, RPT)])
    plsc.subcore_barrier()

    def fire(k, _):
        pltpu.async_copy(w_v.at[k], deg_sh.at[di_v.at[k]], dsem, add=True)
        return 0

    lax.fori_loop(0, NCHUNK_D, fire, 0)

    def drain(k, _):
        pltpu.make_async_copy(w_v.at[0], deg_sh.at[di_v.at[0]], dsem).wait()
        return 0

    lax.fori_loop(0, NCHUNK_D, drain, 0)
    plsc.subcore_barrier()
    pltpu.sync_copy(deg_sh.at[pl.ds(s * RPT, RPT)],
                    out_hbm.at[c, pl.ds(s * RPT, RPT)])


def _sc_edge_body(ep_hbm, ew5_hbm, hs_hbm, out_hbm, e_v, w_v, rows0, rows1,
                  rows2, gs0, gs1, gs2, ss0, ss1, ss2, acc_sh):
    c = lax.axis_index("c")
    s = lax.axis_index("s")
    rows = (rows0, rows1, rows2)
    gs = (gs0, gs1, gs2)
    ss = (ss0, ss1, ss2)
    _zero_rows(rows0, C)
    for i in range(RPT // C):
        pltpu.sync_copy(rows0, acc_sh.at[pl.ds(s * RPT + i * C, C)])
    plsc.subcore_barrier()

    def scale(buf, k):
        def grp(g, _):
            wv = w_v[k, pl.ds(g * 16, 16)]
            for l in range(16):
                wl = wv[l]
                r = g * 16 + l
                for j in range(D // 16):
                    buf[r, pl.ds(j * 16, 16)] = buf[r, pl.ds(j * 16, 16)] * wl
            return 0

        lax.fori_loop(0, C // 16, grp, 0)

    def gstart(k, j):
        pltpu.async_copy(hs_hbm.at[e_v.at[0, k]], rows[j], gs[j])

    def gwait(j):
        pltpu.make_async_copy(hs_hbm.at[e_v.at[0, 0]], rows[j], gs[j]).wait()

    def sstart(k, j):
        pltpu.async_copy(rows[j], acc_sh.at[e_v.at[1, k]], ss[j], add=True)

    def swait(j):
        pltpu.make_async_copy(rows[j], acc_sh.at[e_v.at[1, 0]], ss[j]).wait()

    def step(k, j):
        gwait(j)
        scale(rows[j], k)
        sstart(k, j)
        prv = (j - 1) % 3

        @pl.when(k > 0)
        def _():
            swait(prv)

        @pl.when(k + 2 < KB)
        def _():
            gstart(k + 2, prv)

    def block(b, _):
        pltpu.sync_copy(ep_hbm.at[c, s, b], e_v)
        pltpu.sync_copy(ew5_hbm.at[c, s, b], w_v)
        gstart(0, 0)
        gstart(1, 1)

        def tri(i, _):
            for j in range(3):
                step(3 * i + j, j)
            return 0

        lax.fori_loop(0, KB // 3, tri, 0)
        gwait(0)
        scale(rows0, KB - 1)
        sstart(KB - 1, 0)
        swait(2)
        swait(0)
        return 0

    lax.fori_loop(0, NB, block, 0)
    plsc.subcore_barrier()
    pltpu.sync_copy(acc_sh.at[pl.ds(s * RPT, RPT)],
                    out_hbm.at[c, pl.ds(s * RPT, RPT)])


@functools.cache
def _sc_kernels():
    mesh = plsc.VectorSubcoreMesh(core_axis_name="c", subcore_axis_name="s",
                                  num_cores=NC, num_subcores=NS)
    sc_deg = pl.kernel(
        _sc_deg_body,
        out_type=jax.ShapeDtypeStruct((NC, NPAD), jnp.float32),
        mesh=mesh,
        scratch_types=[
            pltpu.VMEM((NCHUNK_D, C), jnp.int32),
            pltpu.VMEM((NCHUNK_D, C), jnp.float32),
            pltpu.VMEM((RPT,), jnp.float32),
            pltpu.SemaphoreType.DMA,
            pltpu.VMEM_SHARED((NPAD,), jnp.float32),
        ],
    )
    sc_edge = pl.kernel(
        _sc_edge_body,
        out_type=jax.ShapeDtypeStruct((NC, NPAD, D), jnp.float32),
        mesh=mesh,
        scratch_types=[
            pltpu.VMEM((2, KB, C), jnp.int32),
            pltpu.VMEM((KB, C), jnp.float32),
            pltpu.VMEM((C, D), jnp.float32),
            pltpu.VMEM((C, D), jnp.float32),
            pltpu.VMEM((C, D), jnp.float32),
            pltpu.SemaphoreType.DMA,
            pltpu.SemaphoreType.DMA,
            pltpu.SemaphoreType.DMA,
            pltpu.SemaphoreType.DMA,
            pltpu.SemaphoreType.DMA,
            pltpu.SemaphoreType.DMA,
            pltpu.VMEM_SHARED((NPAD, D), jnp.float32),
        ],
    )
    return sc_deg, sc_edge


_RB = 1000
_GRID = N_NODES // _RB

_BLK = pl.BlockSpec((_RB, D), lambda i: (i, 0))
_BLK_ACC = pl.BlockSpec((NC, _RB, D), lambda i: (0, i, 0))
_BLK_DEG = pl.BlockSpec((NC, _RB, 1), lambda i: (0, i, 0))
_BLK_BIAS = pl.BlockSpec((1, D), lambda i: (0, 0))
_BLK_W = pl.BlockSpec((D, D), lambda i: (0, 0))
_OUT_SHAPE = jax.ShapeDtypeStruct((N_NODES, D), jnp.float32)


def _dinv(dp_ref):
    return lax.rsqrt(dp_ref[0] + dp_ref[1] + 1.0)


def _tc_mm_scale_body(x_ref, w_ref, dp_ref, o_ref):
    o_ref[...] = jnp.dot(x_ref[...], w_ref[...],
                         preferred_element_type=jnp.float32) * _dinv(dp_ref)


def _tc_mm_scale(x, W, degp):
    return pl.pallas_call(
        _tc_mm_scale_body,
        grid=(_GRID,),
        in_specs=[_BLK, _BLK_W, _BLK_DEG],
        out_specs=_BLK,
        out_shape=_OUT_SHAPE,
    )(x, W, degp)


def _tc_mid_body(acc_ref, hs_ref, dp_ref, bias_ref, w_ref, o_ref):
    dv = _dinv(dp_ref)
    t = (acc_ref[0] + acc_ref[1] + hs_ref[...]) * dv + bias_ref[...]
    h = jnp.maximum(t, 0.0)
    o_ref[...] = jnp.dot(h, w_ref[...],
                         preferred_element_type=jnp.float32) * dv


def _tc_mid(acc, hs, degp, bias, W):
    return pl.pallas_call(
        _tc_mid_body,
        grid=(_GRID,),
        in_specs=[_BLK_ACC, _BLK, _BLK_DEG, _BLK_BIAS, _BLK_W],
        out_specs=_BLK,
        out_shape=_OUT_SHAPE,
    )(acc, hs, degp, bias, W)


def _tc_out_body(acc_ref, hs_ref, dp_ref, bias_ref, o_ref):
    o_ref[...] = (acc_ref[0] + acc_ref[1] + hs_ref[...]) * _dinv(dp_ref) \
        + bias_ref[...]


def _tc_out(acc, hs, degp, bias):
    return pl.pallas_call(
        _tc_out_body,
        grid=(_GRID,),
        in_specs=[_BLK_ACC, _BLK, _BLK_DEG, _BLK_BIAS],
        out_specs=_BLK,
        out_shape=_OUT_SHAPE,
    )(acc, hs, degp, bias)


def kernel(x, edge_index, edge_weight, W1, b1, W2, b2):
    src = edge_index[0]
    dst = edge_index[1]

    ep = jnp.stack([src.reshape(NC, NS, NB, KB, C),
                    dst.reshape(NC, NS, NB, KB, C)], axis=3)
    ew5 = edge_weight.reshape(NC, NS, NB, KB, C)
    dst4 = dst.reshape(NC, NS, NCHUNK_D, C)
    ew4 = edge_weight.reshape(NC, NS, NCHUNK_D, C)
    _sc_deg, _sc_edge = _sc_kernels()
    degp = _sc_deg(dst4, ew4)
    dp = degp.reshape(NC, NPAD, 1)
    hs1 = _tc_mm_scale(x, W1, dp)
    acc1 = _sc_edge(ep, ew5, hs1)
    hs2 = _tc_mid(acc1, hs1, dp, b1.reshape(1, D), W2)
    acc2 = _sc_edge(ep, ew5, hs2)
    return _tc_out(acc2, hs2, dp, b2.reshape(1, D))


# speedup vs baseline: 1.1670x; 1.0029x over previous
import functools

import jax
import jax.numpy as jnp
from jax import lax
from jax.experimental import pallas as pl
from jax.experimental.pallas import tpu as pltpu
from jax.experimental.pallas import tpu_sc as plsc

N_NODES = 10000
N_EDGES = 320000
D = 128

NC = 2
NS = 16
NPAD = 10240
RPT = NPAD // NS
EPC = N_EDGES // NC
EPT = EPC // NS
C = 80
NCHUNK_D = EPT // C
NCHUNK = EPT // C
KB = 25
NB = NCHUNK // KB

def _zero_rows(buf, nrows):
    zv = jnp.zeros((16,), jnp.float32)

    def body(i, _):
        for j in range(D // 16):
            buf[i, pl.ds(j * 16, 16)] = zv
        return 0

    lax.fori_loop(0, nrows, body, 0)


def _sc_deg_body(dst4_hbm, ew4_hbm, out_hbm, di_v, w_v, zb_v, dsem, deg_sh):
    c = lax.axis_index("c")
    s = lax.axis_index("s")
    pltpu.sync_copy(dst4_hbm.at[c, s], di_v)
    pltpu.sync_copy(ew4_hbm.at[c, s], w_v)
    zv = jnp.zeros((16,), jnp.float32)

    def zb(i, _):
        zb_v[pl.ds(i * 16, 16)] = zv
        return 0

    lax.fori_loop(0, RPT // 16, zb, 0)
    pltpu.sync_copy(zb_v, deg_sh.at[pl.ds(s * RPT, RPT)])
    plsc.subcore_barrier()

    def fire(k, _):
        pltpu.async_copy(w_v.at[k], deg_sh.at[di_v.at[k]], dsem, add=True)
        return 0

    lax.fori_loop(0, NCHUNK_D, fire, 0)

    def drain(k, _):
        pltpu.make_async_copy(w_v.at[0], deg_sh.at[di_v.at[0]], dsem).wait()
        return 0

    lax.fori_loop(0, NCHUNK_D, drain, 0)
    plsc.subcore_barrier()
    pltpu.sync_copy(deg_sh.at[pl.ds(s * RPT, RPT)],
                    out_hbm.at[c, pl.ds(s * RPT, RPT)])


def _sc_edge_body(ep_hbm, ew5_hbm, hs_hbm, out_hbm, e_v, w_v, rows0, rows1,
                  rows2, gs0, gs1, gs2, ss0, ss1, ss2, acc_sh):
    c = lax.axis_index("c")
    s = lax.axis_index("s")
    rows = (rows0, rows1, rows2)
    gs = (gs0, gs1, gs2)
    ss = (ss0, ss1, ss2)
    _zero_rows(rows0, C)
    for i in range(RPT // C):
        pltpu.sync_copy(rows0, acc_sh.at[pl.ds(s * RPT + i * C, C)])
    plsc.subcore_barrier()

    def scale(buf, k):
        def grp(g, _):
            wv = w_v[k, pl.ds(g * 16, 16)]
            for l in range(16):
                wl = wv[l]
                r = g * 16 + l
                for j in range(D // 16):
                    buf[r, pl.ds(j * 16, 16)] = buf[r, pl.ds(j * 16, 16)] * wl
            return 0

        lax.fori_loop(0, C // 16, grp, 0)

    def gstart(k, j):
        pltpu.async_copy(hs_hbm.at[e_v.at[0, k]], rows[j], gs[j])

    def gwait(j):
        pltpu.make_async_copy(hs_hbm.at[e_v.at[0, 0]], rows[j], gs[j]).wait()

    def sstart(k, j):
        pltpu.async_copy(rows[j], acc_sh.at[e_v.at[1, k]], ss[j], add=True)

    def swait(j):
        pltpu.make_async_copy(rows[j], acc_sh.at[e_v.at[1, 0]], ss[j]).wait()

    def step(k, j):
        gwait(j)
        scale(rows[j], k)
        sstart(k, j)
        prv = (j - 1) % 3

        @pl.when(k > 0)
        def _():
            swait(prv)

        @pl.when(k + 2 < KB)
        def _():
            gstart(k + 2, prv)

    def block(b, _):
        pltpu.sync_copy(ep_hbm.at[c, s, b], e_v)
        pltpu.sync_copy(ew5_hbm.at[c, s, b], w_v)
        gstart(0, 0)
        gstart(1, 1)

        def tri(i, _):
            for j in range(3):
                step(3 * i + j, j)
            return 0

        lax.fori_loop(0, KB // 3, tri, 0)
        gwait(0)
        scale(rows0, KB - 1)
        sstart(KB - 1, 0)
        swait(2)
        swait(0)
        return 0

    lax.fori_loop(0, NB, block, 0)
    plsc.subcore_barrier()
    pltpu.sync_copy(acc_sh.at[pl.ds(s * RPT, RPT)],
                    out_hbm.at[c, pl.ds(s * RPT, RPT)])


@functools.cache
def _sc_kernels():
    mesh = plsc.VectorSubcoreMesh(core_axis_name="c", subcore_axis_name="s",
                                  num_cores=NC, num_subcores=NS)
    sc_deg = pl.kernel(
        _sc_deg_body,
        out_type=jax.ShapeDtypeStruct((NC, NPAD), jnp.float32),
        mesh=mesh,
        scratch_types=[
            pltpu.VMEM((NCHUNK_D, C), jnp.int32),
            pltpu.VMEM((NCHUNK_D, C), jnp.float32),
            pltpu.VMEM((RPT,), jnp.float32),
            pltpu.SemaphoreType.DMA,
            pltpu.VMEM_SHARED((NPAD,), jnp.float32),
        ],
    )
    sc_edge = pl.kernel(
        _sc_edge_body,
        out_type=jax.ShapeDtypeStruct((NC, NPAD, D), jnp.float32),
        mesh=mesh,
        scratch_types=[
            pltpu.VMEM((2, KB, C), jnp.int32),
            pltpu.VMEM((KB, C), jnp.float32),
            pltpu.VMEM((C, D), jnp.float32),
            pltpu.VMEM((C, D), jnp.float32),
            pltpu.VMEM((C, D), jnp.float32),
            pltpu.SemaphoreType.DMA,
            pltpu.SemaphoreType.DMA,
            pltpu.SemaphoreType.DMA,
            pltpu.SemaphoreType.DMA,
            pltpu.SemaphoreType.DMA,
            pltpu.SemaphoreType.DMA,
            pltpu.VMEM_SHARED((NPAD, D), jnp.float32),
        ],
    )
    return sc_deg, sc_edge


_RB = 1000
_GRID = N_NODES // _RB

_BLK = pl.BlockSpec((_RB, D), lambda i: (i, 0))
_BLK_ACC = pl.BlockSpec((NC, _RB, D), lambda i: (0, i, 0))
_BLK_DEG = pl.BlockSpec((NC, _RB, 1), lambda i: (0, i, 0))
_BLK_BIAS = pl.BlockSpec((1, D), lambda i: (0, 0))
_BLK_W = pl.BlockSpec((D, D), lambda i: (0, 0))
_OUT_SHAPE = jax.ShapeDtypeStruct((N_NODES, D), jnp.float32)


def _dinv(dp_ref):
    return lax.rsqrt(dp_ref[0] + dp_ref[1] + 1.0)


def _tc_mm_scale_body(x_ref, w_ref, dp_ref, o_ref):
    o_ref[...] = jnp.dot(x_ref[...], w_ref[...],
                         preferred_element_type=jnp.float32) * _dinv(dp_ref)


def _tc_mm_scale(x, W, degp):
    return pl.pallas_call(
        _tc_mm_scale_body,
        grid=(_GRID,),
        in_specs=[_BLK, _BLK_W, _BLK_DEG],
        out_specs=_BLK,
        out_shape=_OUT_SHAPE,
    )(x, W, degp)


def _tc_mid_body(acc_ref, hs_ref, dp_ref, bias_ref, w_ref, o_ref):
    dv = _dinv(dp_ref)
    t = (acc_ref[0] + acc_ref[1] + hs_ref[...]) * dv + bias_ref[...]
    h = jnp.maximum(t, 0.0)
    o_ref[...] = jnp.dot(h, w_ref[...],
                         preferred_element_type=jnp.float32) * dv


def _tc_mid(acc, hs, degp, bias, W):
    return pl.pallas_call(
        _tc_mid_body,
        grid=(_GRID,),
        in_specs=[_BLK_ACC, _BLK, _BLK_DEG, _BLK_BIAS, _BLK_W],
        out_specs=_BLK,
        out_shape=_OUT_SHAPE,
    )(acc, hs, degp, bias, W)


def _tc_out_body(acc_ref, hs_ref, dp_ref, bias_ref, o_ref):
    o_ref[...] = (acc_ref[0] + acc_ref[1] + hs_ref[...]) * _dinv(dp_ref) \
        + bias_ref[...]


def _tc_out(acc, hs, degp, bias):
    return pl.pallas_call(
        _tc_out_body,
        grid=(_GRID,),
        in_specs=[_BLK_ACC, _BLK, _BLK_DEG, _BLK_BIAS],
        out_specs=_BLK,
        out_shape=_OUT_SHAPE,
    )(acc, hs, degp, bias)


def kernel(x, edge_index, edge_weight, W1, b1, W2, b2):
    src = edge_index[0]
    dst = edge_index[1]

    ep = jnp.stack([src.reshape(NC, NS, NB, KB, C),
                    dst.reshape(NC, NS, NB, KB, C)], axis=3)
    ew5 = edge_weight.reshape(NC, NS, NB, KB, C)
    dst4 = dst.reshape(NC, NS, NCHUNK_D, C)
    ew4 = edge_weight.reshape(NC, NS, NCHUNK_D, C)
    _sc_deg, _sc_edge = _sc_kernels()
    degp = _sc_deg(dst4, ew4)
    dp = degp.reshape(NC, NPAD, 1)
    hs1 = _tc_mm_scale(x, W1, dp)
    acc1 = _sc_edge(ep, ew5, hs1)
    hs2 = _tc_mid(acc1, hs1, dp, b1.reshape(1, D), W2)
    acc2 = _sc_edge(ep, ew5, hs2)
    return _tc_out(acc2, hs2, dp, b2.reshape(1, D))
